# baseline (speedup 1.0000x reference)
import functools

import jax
import jax.numpy as jnp
import numpy as np
from jax import lax
from jax.experimental import pallas as pl
from jax.experimental.pallas import tpu as pltpu
from jax.experimental.pallas import tpu_sc as plsc

_B, _I, _H, _C = 8, 224, 224, 96
_R = _B * _I
_NC, _NS = 2, 16
_NW = _NC * _NS
_RPW = _R // _NW
_IPAD = 64
_L = 16
_W = 8

_PERM = np.array([
    183, 138, 166, 19, 76, 158, 219, 118, 143, 54, 189, 149, 90, 30, 7,
    96, 139, 155, 131, 121, 115, 6, 35, 23, 58, 128, 16, 21, 194, 213,
    156, 220, 77, 154, 160, 94, 116, 61, 38, 3, 185, 105, 132, 81, 26,
    32, 64, 37, 56, 51, 2, 193, 122, 63, 133, 52, 20, 89, 202, 95, 44,
    47, 123, 79, 84, 222, 144, 157, 135, 50, 140, 78, 179, 72, 163, 191,
    83, 42, 62, 152, 69, 53, 223, 148, 172, 215, 0, 201, 145, 8, 208,
    203, 167, 169, 159, 109, 181, 22, 178, 13, 29, 99, 110, 34, 70, 175,
    18, 103, 196, 141, 86, 142, 75, 198, 187, 206, 91, 111, 24, 113, 1,
    65, 48, 5, 45, 199, 165, 150, 49, 173, 214, 33, 216, 74, 55, 182,
    136, 60, 204, 119, 57, 124, 27, 112, 129, 209, 151, 10, 134, 192,
    186, 93, 176, 161, 68, 146, 15, 217, 73, 40, 210, 67, 88, 102, 107,
    66, 80, 100, 120, 211, 147, 71, 207, 17, 59, 184, 98, 108, 114, 36,
    125, 101, 218, 180, 92, 171, 153, 28, 46, 9, 104, 200, 117, 221, 4,
    177, 170, 190, 130, 12, 168, 195, 188, 87, 85, 212, 14, 174, 82, 31,
    106, 127, 162, 126, 164, 97, 41, 137, 197, 25, 43, 39, 11, 205,
], dtype=np.int32)


def _index_table() -> np.ndarray:
    r = np.arange(_R)
    gidx = ((r // _I) * _I + _PERM[r % _I]).astype(np.int32)
    gidx = gidx.reshape(_NW, _RPW)
    pad = np.zeros((_NW, _IPAD - _RPW), dtype=np.int32)
    return np.concatenate([gidx, pad], axis=1).reshape(_NW * _IPAD)


def _make_sc_shuffle():
    mesh = plsc.VectorSubcoreMesh(
        core_axis_name="c", subcore_axis_name="s",
        num_cores=_NC, num_subcores=_NS,
    )

    @functools.partial(
        pl.kernel,
        out_type=jax.ShapeDtypeStruct((_R, _C, _H), jnp.float32),
        mesh=mesh,
        compiler_params=pltpu.CompilerParams(
            use_tc_tiling_on_sc=True, needs_layout_passes=False
        ),
        scratch_types=[
            pltpu.VMEM((_IPAD,), jnp.int32),
            pltpu.SemaphoreType.DMA,
        ],
    )
    def shuffle(in_hbm, idx_hbm, out_hbm, idx_v, sem):
        wid = lax.axis_index("s") * _NC + lax.axis_index("c")
        base = wid * _RPW
        pltpu.sync_copy(idx_hbm.at[pl.ds(wid * _IPAD, _IPAD)], idx_v)
        lanes = lax.iota(jnp.int32, _L)

        def _fire(g):
            word = pl.multiple_of((g // _L) * _L, _L)
            vec = idx_v[pl.ds(word, _L)]
            sel = jnp.where(lanes == g % _L, vec, 0)
            src = lax.reduce_max(sel, axes=(0,))
            pltpu.async_copy(in_hbm.at[src], out_hbm.at[base + g], sem)

        def _drain_one():
            pltpu.make_async_copy(
                in_hbm.at[base], out_hbm.at[base], sem
            ).wait()

        @pl.loop(0, _RPW, step=4)
        def _slabs(g0):
            for b in range(4):
                g = g0 + b
                _fire(g)

                @pl.when(g >= _W)
                def _():
                    _drain_one()

        for _ in range(_W):
            _drain_one()

    return shuffle


def kernel(inputs):
    x3 = jnp.transpose(inputs, (0, 1, 3, 2)).reshape(_R, _C, _H)
    idx = jnp.asarray(_index_table())
    out3 = _make_sc_shuffle()(x3, idx)
    return jnp.transpose(out3.reshape(_B, _I, _C, _H), (0, 1, 3, 2))


# speedup vs baseline: 37.9334x; 37.9334x over previous
import functools

import jax
import jax.numpy as jnp
import numpy as np
from jax import lax
from jax.experimental import pallas as pl
from jax.experimental.pallas import tpu as pltpu
from jax.experimental.pallas import tpu_sc as plsc

_B, _I, _H, _C = 8, 224, 224, 96
_R = _B * _I
_NC, _NS = 2, 16
_NW = _NC * _NS
_RPW = _R // _NW
_IPAD = 64
_L = 16
_CH = _C // 2
_U = _RPW * 2
_NBUF = 8
_LOOK = 4

_PERM = np.array([
    183, 138, 166, 19, 76, 158, 219, 118, 143, 54, 189, 149, 90, 30, 7,
    96, 139, 155, 131, 121, 115, 6, 35, 23, 58, 128, 16, 21, 194, 213,
    156, 220, 77, 154, 160, 94, 116, 61, 38, 3, 185, 105, 132, 81, 26,
    32, 64, 37, 56, 51, 2, 193, 122, 63, 133, 52, 20, 89, 202, 95, 44,
    47, 123, 79, 84, 222, 144, 157, 135, 50, 140, 78, 179, 72, 163, 191,
    83, 42, 62, 152, 69, 53, 223, 148, 172, 215, 0, 201, 145, 8, 208,
    203, 167, 169, 159, 109, 181, 22, 178, 13, 29, 99, 110, 34, 70, 175,
    18, 103, 196, 141, 86, 142, 75, 198, 187, 206, 91, 111, 24, 113, 1,
    65, 48, 5, 45, 199, 165, 150, 49, 173, 214, 33, 216, 74, 55, 182,
    136, 60, 204, 119, 57, 124, 27, 112, 129, 209, 151, 10, 134, 192,
    186, 93, 176, 161, 68, 146, 15, 217, 73, 40, 210, 67, 88, 102, 107,
    66, 80, 100, 120, 211, 147, 71, 207, 17, 59, 184, 98, 108, 114, 36,
    125, 101, 218, 180, 92, 171, 153, 28, 46, 9, 104, 200, 117, 221, 4,
    177, 170, 190, 130, 12, 168, 195, 188, 87, 85, 212, 14, 174, 82, 31,
    106, 127, 162, 126, 164, 97, 41, 137, 197, 25, 43, 39, 11, 205,
], dtype=np.int32)


def _index_table() -> np.ndarray:
    r = np.arange(_R)
    gidx = ((r // _I) * _I + _PERM[r % _I]).astype(np.int32)
    gidx = gidx.reshape(_NW, _RPW)
    pad = np.zeros((_NW, _IPAD - _RPW), dtype=np.int32)
    return np.concatenate([gidx, pad], axis=1).reshape(_NW * _IPAD)


def _make_sc_shuffle():
    mesh = plsc.VectorSubcoreMesh(
        core_axis_name="c", subcore_axis_name="s",
        num_cores=_NC, num_subcores=_NS,
    )

    @functools.partial(
        pl.kernel,
        out_type=jax.ShapeDtypeStruct((_R, _C, _H), jnp.float32),
        mesh=mesh,
        compiler_params=pltpu.CompilerParams(
            use_tc_tiling_on_sc=True, needs_layout_passes=False
        ),
        scratch_types=[
            pltpu.VMEM((_IPAD,), jnp.int32),
            pltpu.VMEM((_NBUF, _CH, _H), jnp.float32),
            [pltpu.SemaphoreType.DMA] * _NBUF,
            [pltpu.SemaphoreType.DMA] * _NBUF,
        ],
    )
    def shuffle(in_hbm, idx_hbm, out_hbm, idx_v, buf, sem_gs, sem_ws):
        wid = lax.axis_index("s") * _NC + lax.axis_index("c")
        base = wid * _RPW
        pltpu.sync_copy(idx_hbm.at[pl.ds(wid * _IPAD, _IPAD)], idx_v)
        lanes = lax.iota(jnp.int32, _L)

        def _fire_gather(u0, j, b):
            g = u0 // 2 + j // 2
            h = (j % 2) * _CH
            word = pl.multiple_of((g // _L) * _L, _L)
            vec = idx_v[pl.ds(word, _L)]
            sel = jnp.where(lanes == g % _L, vec, 0)
            src = lax.reduce_max(sel, axes=(0,))
            pltpu.async_copy(
                in_hbm.at[src, pl.ds(h, _CH)], buf.at[b], sem_gs[b]
            )

        for b in range(_LOOK):
            _fire_gather(0, b, b)

        @pl.loop(0, _U, step=_NBUF)
        def _group(u0):
            for j in range(_NBUF):
                u = u0 + j
                nj = j + _LOOK
                nb = (j + _LOOK) % _NBUF

                @pl.when(u + _LOOK < _U)
                def _():
                    @pl.when(u + _LOOK >= _NBUF)
                    def _():
                        pltpu.make_async_copy(
                            buf.at[nb], out_hbm.at[base, pl.ds(0, _CH)],
                            sem_ws[nb]
                        ).wait()

                    _fire_gather(u0, nj, nb)

                pltpu.make_async_copy(
                    in_hbm.at[base, pl.ds(0, _CH)], buf.at[j], sem_gs[j]
                ).wait()
                g = u0 // 2 + j // 2
                h = (j % 2) * _CH
                pltpu.async_copy(
                    buf.at[j], out_hbm.at[base + g, pl.ds(h, _CH)],
                    sem_ws[j]
                )

        for b in range(_NBUF):
            pltpu.make_async_copy(
                buf.at[b], out_hbm.at[base, pl.ds(0, _CH)], sem_ws[b]
            ).wait()

    return shuffle


def kernel(inputs):
    x3 = jnp.transpose(inputs, (0, 1, 3, 2)).reshape(_R, _C, _H)
    idx = jnp.asarray(_index_table())
    out3 = _make_sc_shuffle()(x3, idx)
    return jnp.transpose(out3.reshape(_B, _I, _C, _H), (0, 1, 3, 2))
